# Initial kernel scaffold; baseline (speedup 1.0000x reference)
import jax
import jax.numpy as jnp
from jax.experimental import pallas as pl


def kernel(x, edge_idx, W, b, gamma, beta):
    raise NotImplementedError("write your pallas kernel here")


# speedup vs baseline: 11.2192x; 11.2192x over previous
import functools

import jax
import jax.numpy as jnp
from jax import lax
from jax.experimental import pallas as pl
from jax.experimental.pallas import tpu as pltpu
from jax.experimental.pallas import tpu_sc as plsc

N = 10000
E = 320000
D = 128
EPS = 1e-5
ALPHA = 0.01

NC = 2
NS = 16
NW = NC * NS
EPW = E // NW
CH = 128
NFULL = EPW // CH
TAIL = EPW - NFULL * CH
TRASH = N
RPT = 640
NACC = NS * RPT

_mesh = plsc.VectorSubcoreMesh(core_axis_name="c", subcore_axis_name="s")


@functools.partial(
    pl.kernel,
    mesh=_mesh,
    out_type=(
        jax.ShapeDtypeStruct((NC * NACC,), jnp.float32),
        jax.ShapeDtypeStruct((E,), jnp.int32),
    ),
    scratch_types=[
        pltpu.VMEM((CH,), jnp.int32),
        pltpu.VMEM((CH,), jnp.int32),
        pltpu.VMEM((CH,), jnp.int32),
        pltpu.VMEM((CH,), jnp.float32),
        pltpu.VMEM((TAIL,), jnp.int32),
        pltpu.VMEM((TAIL,), jnp.int32),
        pltpu.VMEM((TAIL,), jnp.int32),
        pltpu.VMEM((TAIL,), jnp.float32),
        pltpu.VMEM((RPT,), jnp.float32),
        pltpu.VMEM_SHARED((NACC,), jnp.float32),
    ],
)
def _edge_prep(src_h, dst_h, degp_h, dstp_h,
               srcc, dstc, ndc, valc, srct, dstt, ndt, valt, zb, deg_sh):
    cid = lax.axis_index("c")
    sid = lax.axis_index("s")
    wid = sid * NC + cid

    for j in range(RPT // 16):
        zb[pl.ds(j * 16, 16)] = jnp.zeros((16,), jnp.float32)
    pltpu.sync_copy(zb, deg_sh.at[pl.ds(sid * RPT, RPT)])
    plsc.subcore_barrier()

    ebase = wid * EPW

    def body(i, carry):
        base = ebase + i * CH
        pltpu.sync_copy(src_h.at[pl.ds(base, CH)], srcc)
        pltpu.sync_copy(dst_h.at[pl.ds(base, CH)], dstc)
        for j in range(CH // 16):
            sl = pl.ds(j * 16, 16)
            s16 = srcc[sl]
            d16 = dstc[sl]
            m = s16 == d16
            ndc[sl] = jnp.where(m, TRASH, d16)
            valc[sl] = jnp.where(m, 0.0, 1.0)
        pltpu.sync_copy(ndc, dstp_h.at[pl.ds(base, CH)])
        pltpu.sync_copy(valc, deg_sh.at[srcc], add=True)
        return carry

    lax.fori_loop(0, NFULL, body, 0)

    base = ebase + NFULL * CH
    pltpu.sync_copy(src_h.at[pl.ds(base, TAIL)], srct)
    pltpu.sync_copy(dst_h.at[pl.ds(base, TAIL)], dstt)
    s16 = srct[...]
    d16 = dstt[...]
    m = s16 == d16
    ndt[...] = jnp.where(m, TRASH, d16)
    valt[...] = jnp.where(m, 0.0, 1.0)
    pltpu.sync_copy(ndt, dstp_h.at[pl.ds(base, TAIL)])
    pltpu.sync_copy(valt, deg_sh.at[srct], add=True)

    plsc.subcore_barrier()
    pltpu.sync_copy(
        deg_sh.at[pl.ds(sid * RPT, RPT)],
        degp_h.at[pl.ds(cid * NACC + sid * RPT, RPT)],
    )


@functools.partial(
    pl.kernel,
    mesh=_mesh,
    out_type=jax.ShapeDtypeStruct((NC * NACC, D), jnp.float32),
    scratch_types=[
        pltpu.VMEM((CH,), jnp.int32),
        pltpu.VMEM((CH,), jnp.int32),
        pltpu.VMEM((CH, D), jnp.float32),
        pltpu.VMEM((TAIL,), jnp.int32),
        pltpu.VMEM((TAIL,), jnp.int32),
        pltpu.VMEM((TAIL, D), jnp.float32),
        pltpu.VMEM((8, D), jnp.float32),
        pltpu.VMEM_SHARED((NACC, D), jnp.float32),
        pltpu.SemaphoreType.DMA,
    ],
)
def _spmv(g_h, src_h, dstp_h, out_h,
          srcc, dstc, rows, srct, dstt, rowst, zb, acc_sh, sem):
    cid = lax.axis_index("c")
    sid = lax.axis_index("s")
    wid = sid * NC + cid

    for r in range(8):
        for j in range(D // 16):
            zb[r, pl.ds(j * 16, 16)] = jnp.zeros((16,), jnp.float32)

    def zbody(i, carry):
        pltpu.sync_copy(zb, acc_sh.at[pl.ds(sid * RPT + i * 8, 8)])
        return carry

    lax.fori_loop(0, RPT // 8, zbody, 0)
    plsc.subcore_barrier()

    ebase = wid * EPW

    def body(i, carry):
        base = ebase + i * CH
        pltpu.sync_copy(src_h.at[pl.ds(base, CH)], srcc)
        pltpu.sync_copy(dstp_h.at[pl.ds(base, CH)], dstc)
        pltpu.async_copy(g_h.at[srcc], rows, sem).wait()
        pltpu.sync_copy(rows, acc_sh.at[dstc], add=True)
        return carry

    lax.fori_loop(0, NFULL, body, 0)

    base = ebase + NFULL * CH
    pltpu.sync_copy(src_h.at[pl.ds(base, TAIL)], srct)
    pltpu.sync_copy(dstp_h.at[pl.ds(base, TAIL)], dstt)
    pltpu.async_copy(g_h.at[srct], rowst, sem).wait()
    pltpu.sync_copy(rowst, acc_sh.at[dstt], add=True)

    plsc.subcore_barrier()
    pltpu.sync_copy(
        acc_sh.at[pl.ds(sid * RPT, RPT)],
        out_h.at[pl.ds(cid * NACC + sid * RPT, RPT)],
    )


BLK = 1000
GRID = N // BLK


def _prescale_body(degT_ref, x_ref, dinv_ref, g_ref):
    deg = degT_ref[...]
    d = deg[:, 0:1] + deg[:, 1:2]
    dinv = jnp.where(d > 0.0, lax.rsqrt(d), 0.0)
    dinv_ref[...] = dinv
    g_ref[...] = x_ref[...] * dinv


def _prescale(degT, x):
    return pl.pallas_call(
        _prescale_body,
        grid=(GRID,),
        in_specs=[
            pl.BlockSpec((BLK, NC), lambda i: (i, 0)),
            pl.BlockSpec((BLK, D), lambda i: (i, 0)),
        ],
        out_specs=[
            pl.BlockSpec((BLK, 1), lambda i: (i, 0)),
            pl.BlockSpec((BLK, D), lambda i: (i, 0)),
        ],
        out_shape=[
            jax.ShapeDtypeStruct((N, 1), jnp.float32),
            jax.ShapeDtypeStruct((N, D), jnp.float32),
        ],
    )(degT, x)


def _mid_body(dinv_ref, s1a_ref, s1b_ref, g2_ref):
    d = dinv_ref[...]
    g2_ref[...] = -(d * d) * (s1a_ref[...] + s1b_ref[...])


def _mid(dinv, sa, sb):
    return pl.pallas_call(
        _mid_body,
        grid=(GRID,),
        in_specs=[
            pl.BlockSpec((BLK, 1), lambda i: (i, 0)),
            pl.BlockSpec((BLK, D), lambda i: (i, 0)),
            pl.BlockSpec((BLK, D), lambda i: (i, 0)),
        ],
        out_specs=pl.BlockSpec((BLK, D), lambda i: (i, 0)),
        out_shape=jax.ShapeDtypeStruct((N, D), jnp.float32),
    )(dinv, sa, sb)


def _mm_body(x_ref, dinv_ref, s1a, s1b, s2a, s2b, w_ref, b_ref,
             out_ref, st_ref, acc):
    i = pl.program_id(0)
    d = dinv_ref[...]
    xv = x_ref[...]
    tx1 = -d * (s1a[...] + s1b[...])
    tx2 = -2.0 * d * (s2a[...] + s2b[...]) - xv
    o = (jnp.dot(xv, w_ref[0], preferred_element_type=jnp.float32)
         + jnp.dot(tx1, w_ref[1], preferred_element_type=jnp.float32)
         + jnp.dot(tx2, w_ref[2], preferred_element_type=jnp.float32)
         + b_ref[...])
    out_ref[...] = o

    @pl.when(i == 0)
    def _init():
        acc[...] = jnp.zeros_like(acc)

    acc[0:1, :] += jnp.sum(o, axis=0, keepdims=True)
    acc[1:2, :] += jnp.sum(o * o, axis=0, keepdims=True)

    @pl.when(i == GRID - 1)
    def _fin():
        st_ref[...] = acc[...]


def _mm(x, dinv, s1a, s1b, s2a, s2b, W, b2):
    return pl.pallas_call(
        _mm_body,
        grid=(GRID,),
        in_specs=[
            pl.BlockSpec((BLK, D), lambda i: (i, 0)),
            pl.BlockSpec((BLK, 1), lambda i: (i, 0)),
            pl.BlockSpec((BLK, D), lambda i: (i, 0)),
            pl.BlockSpec((BLK, D), lambda i: (i, 0)),
            pl.BlockSpec((BLK, D), lambda i: (i, 0)),
            pl.BlockSpec((BLK, D), lambda i: (i, 0)),
            pl.BlockSpec((3, D, D), lambda i: (0, 0, 0)),
            pl.BlockSpec((1, D), lambda i: (0, 0)),
        ],
        out_specs=[
            pl.BlockSpec((BLK, D), lambda i: (i, 0)),
            pl.BlockSpec((2, D), lambda i: (0, 0)),
        ],
        out_shape=[
            jax.ShapeDtypeStruct((N, D), jnp.float32),
            jax.ShapeDtypeStruct((2, D), jnp.float32),
        ],
        scratch_shapes=[pltpu.VMEM((2, D), jnp.float32)],
    )(x, dinv, s1a, s1b, s2a, s2b, W, b2)


def _norm_body(o_ref, st_ref, gam_ref, bet_ref, y_ref):
    st = st_ref[...]
    mean = st[0:1, :] * (1.0 / N)
    var = st[1:2, :] * (1.0 / N) - mean * mean
    scale = lax.rsqrt(var + EPS) * gam_ref[...]
    y = (o_ref[...] - mean) * scale + bet_ref[...]
    y_ref[...] = jnp.where(y >= 0.0, y, ALPHA * y)


def _norm(o, st, gam2, bet2):
    return pl.pallas_call(
        _norm_body,
        grid=(GRID,),
        in_specs=[
            pl.BlockSpec((BLK, D), lambda i: (i, 0)),
            pl.BlockSpec((2, D), lambda i: (0, 0)),
            pl.BlockSpec((1, D), lambda i: (0, 0)),
            pl.BlockSpec((1, D), lambda i: (0, 0)),
        ],
        out_specs=pl.BlockSpec((BLK, D), lambda i: (i, 0)),
        out_shape=jax.ShapeDtypeStruct((N, D), jnp.float32),
    )(o, st, gam2, bet2)


@jax.jit
def kernel(x, edge_idx, W, b, gamma, beta):
    src = edge_idx[0]
    dst = edge_idx[1]
    degp, dstp = _edge_prep(src, dst)
    degT = jnp.transpose(degp.reshape(NC, NACC)[:, :N])
    dinv, g1 = _prescale(degT, x)
    s1 = _spmv(g1, src, dstp)
    s1a, s1b = s1[:N], s1[NACC:NACC + N]
    g2 = _mid(dinv, s1a, s1b)
    s2 = _spmv(g2, src, dstp)
    s2a, s2b = s2[:N], s2[NACC:NACC + N]
    outp, stats = _mm(x, dinv, s1a, s1b, s2a, s2b, W, b.reshape(1, D))
    return _norm(outp, stats, gamma.reshape(1, D), beta.reshape(1, D))


# speedup vs baseline: 22.6902x; 2.0224x over previous
import functools

import jax
import jax.numpy as jnp
from jax import lax
from jax.experimental import pallas as pl
from jax.experimental.pallas import tpu as pltpu
from jax.experimental.pallas import tpu_sc as plsc

N = 10000
E = 320000
D = 128
EPS = 1e-5
ALPHA = 0.01

NC = 2
NS = 16
NW = NC * NS
EPW = E // NW
ECH = 128
ENFULL = EPW // ECH
CH = 96
NFULL = EPW // CH
TAIL = EPW - NFULL * CH
TRASH = N
RPT = 640
NACC = NS * RPT

_mesh = plsc.VectorSubcoreMesh(core_axis_name="c", subcore_axis_name="s")


@functools.partial(
    pl.kernel,
    mesh=_mesh,
    out_type=(
        jax.ShapeDtypeStruct((NC * NACC,), jnp.float32),
        jax.ShapeDtypeStruct((E,), jnp.int32),
    ),
    scratch_types=[
        pltpu.VMEM((EPW,), jnp.int32),
        pltpu.VMEM((EPW,), jnp.int32),
        pltpu.VMEM((EPW,), jnp.int32),
        pltpu.VMEM((ECH,), jnp.int32),
        pltpu.VMEM((ECH,), jnp.int32),
        pltpu.VMEM((ECH,), jnp.float32),
        pltpu.VMEM((ECH,), jnp.float32),
        pltpu.VMEM((TAIL,), jnp.int32),
        pltpu.VMEM((TAIL,), jnp.float32),
        pltpu.VMEM((RPT,), jnp.float32),
        pltpu.VMEM_SHARED((NACC,), jnp.float32),
        pltpu.SemaphoreType.DMA,
        pltpu.SemaphoreType.DMA,
    ],
)
def _edge_prep(src_h, dst_h, degp_h, dstp_h,
               srcb, dstb, ndb, ic0, ic1, vc0, vc1, srct, valt, zb, deg_sh,
               s0, s1):
    cid = lax.axis_index("c")
    sid = lax.axis_index("s")
    wid = sid * NC + cid
    ebase = wid * EPW

    pltpu.async_copy(src_h.at[pl.ds(ebase, EPW)], srcb, s0)
    pltpu.async_copy(dst_h.at[pl.ds(ebase, EPW)], dstb, s1)
    for j in range(RPT // 16):
        zb[pl.ds(j * 16, 16)] = jnp.zeros((16,), jnp.float32)
    pltpu.sync_copy(zb, deg_sh.at[pl.ds(sid * RPT, RPT)])
    pltpu.make_async_copy(src_h.at[pl.ds(ebase, EPW)], srcb, s0).wait()
    pltpu.make_async_copy(dst_h.at[pl.ds(ebase, EPW)], dstb, s1).wait()
    plsc.subcore_barrier()

    bufs = ((ic0, vc0, s0), (ic1, vc1, s1))

    def compute(i, ic, vc):
        off = i * ECH
        for k in range(ECH // 16):
            sl = pl.ds(k * 16, 16)
            dsl = pl.ds(off + k * 16, 16)
            s16 = srcb[dsl]
            d16 = dstb[dsl]
            m = s16 == d16
            ndb[dsl] = jnp.where(m, TRASH, d16)
            ic[sl] = s16
            vc[sl] = jnp.where(m, 0.0, 1.0)

    for b in range(2):
        ic, vc, sem = bufs[b]
        compute(b, ic, vc)
        pltpu.async_copy(vc, deg_sh.at[ic], sem, add=True)

    def body(j, carry):
        for b in range(2):
            i = 2 * j + 2 + b
            ic, vc, sem = bufs[b]
            pltpu.make_async_copy(vc, deg_sh.at[ic], sem).wait()
            compute(i, ic, vc)
            pltpu.async_copy(vc, deg_sh.at[ic], sem, add=True)
        return carry

    lax.fori_loop(0, ENFULL // 2 - 1, body, 0)
    for b in range(2):
        ic, vc, sem = bufs[b]
        pltpu.make_async_copy(vc, deg_sh.at[ic], sem).wait()

    off = ENFULL * ECH
    dsl = pl.ds(off, 16)
    s16 = srcb[dsl]
    d16 = dstb[dsl]
    m = s16 == d16
    ndb[dsl] = jnp.where(m, TRASH, d16)
    srct[...] = s16
    valt[...] = jnp.where(m, 0.0, 1.0)
    pltpu.sync_copy(valt, deg_sh.at[srct], add=True)

    pltpu.sync_copy(ndb, dstp_h.at[pl.ds(ebase, EPW)])

    plsc.subcore_barrier()
    pltpu.sync_copy(
        deg_sh.at[pl.ds(sid * RPT, RPT)],
        degp_h.at[pl.ds(cid * NACC + sid * RPT, RPT)],
    )


@functools.partial(
    pl.kernel,
    mesh=_mesh,
    out_type=jax.ShapeDtypeStruct((NC * NACC, D), jnp.float32),
    scratch_types=[
        pltpu.VMEM((EPW,), jnp.int32),
        pltpu.VMEM((EPW,), jnp.int32),
        pltpu.VMEM((CH,), jnp.int32),
        pltpu.VMEM((CH,), jnp.int32),
        pltpu.VMEM((CH,), jnp.int32),
        pltpu.VMEM((CH,), jnp.int32),
        pltpu.VMEM((CH, D), jnp.float32),
        pltpu.VMEM((CH, D), jnp.float32),
        pltpu.VMEM((TAIL,), jnp.int32),
        pltpu.VMEM((TAIL,), jnp.int32),
        pltpu.VMEM((TAIL, D), jnp.float32),
        pltpu.VMEM((8, D), jnp.float32),
        pltpu.VMEM_SHARED((NACC, D), jnp.float32),
        pltpu.SemaphoreType.DMA,
        pltpu.SemaphoreType.DMA,
    ],
)
def _spmv(g_h, src_h, dstp_h, out_h,
          srcb, dstb, sc0, sc1, dc0, dc1, r0, r1,
          srct, dstt, rowst, zb, acc_sh, sg0, sg1):
    cid = lax.axis_index("c")
    sid = lax.axis_index("s")
    wid = sid * NC + cid
    ebase = wid * EPW

    for r in range(8):
        for j in range(D // 16):
            zb[r, pl.ds(j * 16, 16)] = jnp.zeros((16,), jnp.float32)

    pltpu.async_copy(src_h.at[pl.ds(ebase, EPW)], srcb, sg0)
    pltpu.async_copy(dstp_h.at[pl.ds(ebase, EPW)], dstb, sg1)

    def zbody(i, carry):
        pltpu.sync_copy(zb, acc_sh.at[pl.ds(sid * RPT + i * 8, 8)])
        return carry

    lax.fori_loop(0, RPT // 8, zbody, 0)
    pltpu.make_async_copy(src_h.at[pl.ds(ebase, EPW)], srcb, sg0).wait()
    pltpu.make_async_copy(dstp_h.at[pl.ds(ebase, EPW)], dstb, sg1).wait()
    plsc.subcore_barrier()

    bufs = ((sc0, dc0, r0, sg0), (sc1, dc1, r1, sg1))

    def fill(i, sc, dc):
        off = i * CH
        for k in range(CH // 16):
            sl = pl.ds(k * 16, 16)
            sc[sl] = srcb[pl.ds(off + k * 16, 16)]
            dc[sl] = dstb[pl.ds(off + k * 16, 16)]

    for b in range(2):
        sc, dc, rw, sg = bufs[b]
        fill(b, sc, dc)
        pltpu.async_copy(g_h.at[sc], rw, sg)

    def body(j, carry):
        for b in range(2):
            i = 2 * j + b
            sc, dc, rw, sg = bufs[b]
            pltpu.make_async_copy(g_h.at[sc], rw, sg).wait()
            pltpu.sync_copy(rw, acc_sh.at[dc], add=True)
            fill(i + 2, sc, dc)
            pltpu.async_copy(g_h.at[sc], rw, sg)
        return carry

    lax.fori_loop(0, NFULL // 2 - 1, body, 0)

    for b in range(2):
        sc, dc, rw, sg = bufs[b]
        pltpu.make_async_copy(g_h.at[sc], rw, sg).wait()
        pltpu.sync_copy(rw, acc_sh.at[dc], add=True)

    off = NFULL * CH
    for k in range(TAIL // 16):
        sl = pl.ds(k * 16, 16)
        srct[sl] = srcb[pl.ds(off + k * 16, 16)]
        dstt[sl] = dstb[pl.ds(off + k * 16, 16)]
    pltpu.async_copy(g_h.at[srct], rowst, sg0).wait()
    pltpu.sync_copy(rowst, acc_sh.at[dstt], add=True)

    plsc.subcore_barrier()
    pltpu.sync_copy(
        acc_sh.at[pl.ds(sid * RPT, RPT)],
        out_h.at[pl.ds(cid * NACC + sid * RPT, RPT)],
    )


BLK = 1000
GRID = N // BLK


def _prescale_body(degT_ref, x_ref, dinv_ref, g_ref):
    deg = degT_ref[...]
    d = deg[:, 0:1] + deg[:, 1:2]
    dinv = jnp.where(d > 0.0, lax.rsqrt(d), 0.0)
    dinv_ref[...] = dinv
    g_ref[...] = x_ref[...] * dinv


def _prescale(degT, x):
    return pl.pallas_call(
        _prescale_body,
        grid=(GRID,),
        in_specs=[
            pl.BlockSpec((BLK, NC), lambda i: (i, 0)),
            pl.BlockSpec((BLK, D), lambda i: (i, 0)),
        ],
        out_specs=[
            pl.BlockSpec((BLK, 1), lambda i: (i, 0)),
            pl.BlockSpec((BLK, D), lambda i: (i, 0)),
        ],
        out_shape=[
            jax.ShapeDtypeStruct((N, 1), jnp.float32),
            jax.ShapeDtypeStruct((N, D), jnp.float32),
        ],
    )(degT, x)


def _mid_body(dinv_ref, s1a_ref, s1b_ref, g2_ref):
    d = dinv_ref[...]
    g2_ref[...] = -(d * d) * (s1a_ref[...] + s1b_ref[...])


def _mid(dinv, sa, sb):
    return pl.pallas_call(
        _mid_body,
        grid=(GRID,),
        in_specs=[
            pl.BlockSpec((BLK, 1), lambda i: (i, 0)),
            pl.BlockSpec((BLK, D), lambda i: (i, 0)),
            pl.BlockSpec((BLK, D), lambda i: (i, 0)),
        ],
        out_specs=pl.BlockSpec((BLK, D), lambda i: (i, 0)),
        out_shape=jax.ShapeDtypeStruct((N, D), jnp.float32),
    )(dinv, sa, sb)


def _mm_body(x_ref, dinv_ref, s1a, s1b, s2a, s2b, w_ref, b_ref,
             out_ref, st_ref, acc):
    i = pl.program_id(0)
    d = dinv_ref[...]
    xv = x_ref[...]
    tx1 = -d * (s1a[...] + s1b[...])
    tx2 = -2.0 * d * (s2a[...] + s2b[...]) - xv
    o = (jnp.dot(xv, w_ref[0], preferred_element_type=jnp.float32)
         + jnp.dot(tx1, w_ref[1], preferred_element_type=jnp.float32)
         + jnp.dot(tx2, w_ref[2], preferred_element_type=jnp.float32)
         + b_ref[...])
    out_ref[...] = o

    @pl.when(i == 0)
    def _init():
        acc[...] = jnp.zeros_like(acc)

    acc[0:1, :] += jnp.sum(o, axis=0, keepdims=True)
    acc[1:2, :] += jnp.sum(o * o, axis=0, keepdims=True)

    @pl.when(i == GRID - 1)
    def _fin():
        st_ref[...] = acc[...]


def _mm(x, dinv, s1a, s1b, s2a, s2b, W, b2):
    return pl.pallas_call(
        _mm_body,
        grid=(GRID,),
        in_specs=[
            pl.BlockSpec((BLK, D), lambda i: (i, 0)),
            pl.BlockSpec((BLK, 1), lambda i: (i, 0)),
            pl.BlockSpec((BLK, D), lambda i: (i, 0)),
            pl.BlockSpec((BLK, D), lambda i: (i, 0)),
            pl.BlockSpec((BLK, D), lambda i: (i, 0)),
            pl.BlockSpec((BLK, D), lambda i: (i, 0)),
            pl.BlockSpec((3, D, D), lambda i: (0, 0, 0)),
            pl.BlockSpec((1, D), lambda i: (0, 0)),
        ],
        out_specs=[
            pl.BlockSpec((BLK, D), lambda i: (i, 0)),
            pl.BlockSpec((2, D), lambda i: (0, 0)),
        ],
        out_shape=[
            jax.ShapeDtypeStruct((N, D), jnp.float32),
            jax.ShapeDtypeStruct((2, D), jnp.float32),
        ],
        scratch_shapes=[pltpu.VMEM((2, D), jnp.float32)],
    )(x, dinv, s1a, s1b, s2a, s2b, W, b2)


def _norm_body(o_ref, st_ref, gam_ref, bet_ref, y_ref):
    st = st_ref[...]
    mean = st[0:1, :] * (1.0 / N)
    var = st[1:2, :] * (1.0 / N) - mean * mean
    scale = lax.rsqrt(var + EPS) * gam_ref[...]
    y = (o_ref[...] - mean) * scale + bet_ref[...]
    y_ref[...] = jnp.where(y >= 0.0, y, ALPHA * y)


def _norm(o, st, gam2, bet2):
    return pl.pallas_call(
        _norm_body,
        grid=(GRID,),
        in_specs=[
            pl.BlockSpec((BLK, D), lambda i: (i, 0)),
            pl.BlockSpec((2, D), lambda i: (0, 0)),
            pl.BlockSpec((1, D), lambda i: (0, 0)),
            pl.BlockSpec((1, D), lambda i: (0, 0)),
        ],
        out_specs=pl.BlockSpec((BLK, D), lambda i: (i, 0)),
        out_shape=jax.ShapeDtypeStruct((N, D), jnp.float32),
    )(o, st, gam2, bet2)


@jax.jit
def kernel(x, edge_idx, W, b, gamma, beta):
    src = edge_idx[0]
    dst = edge_idx[1]
    degp, dstp = _edge_prep(src, dst)
    degT = jnp.transpose(degp.reshape(NC, NACC)[:, :N])
    dinv, g1 = _prescale(degT, x)
    s1 = _spmv(g1, src, dstp)
    s1a, s1b = s1[:N], s1[NACC:NACC + N]
    g2 = _mid(dinv, s1a, s1b)
    s2 = _spmv(g2, src, dstp)
    s2a, s2b = s2[:N], s2[NACC:NACC + N]
    outp, stats = _mm(x, dinv, s1a, s1b, s2a, s2b, W, b.reshape(1, D))
    return _norm(outp, stats, gamma.reshape(1, D), beta.reshape(1, D))


# speedup vs baseline: 24.5299x; 1.0811x over previous
import functools

import jax
import jax.numpy as jnp
from jax import lax
from jax.experimental import pallas as pl
from jax.experimental.pallas import tpu as pltpu
from jax.experimental.pallas import tpu_sc as plsc

N = 10000
E = 320000
D = 128
EPS = 1e-5
ALPHA = 0.01

NC = 2
NS = 16
NW = NC * NS
EPW = E // NW
ECH = 128
ENFULL = EPW // ECH
CH = 64
NFULL = EPW // CH
TAIL = EPW - NFULL * CH
TRASH = N
RPT = 640
NACC = NS * RPT

_mesh = plsc.VectorSubcoreMesh(core_axis_name="c", subcore_axis_name="s")


@functools.partial(
    pl.kernel,
    mesh=_mesh,
    out_type=(
        jax.ShapeDtypeStruct((NC * NACC,), jnp.float32),
        jax.ShapeDtypeStruct((E,), jnp.int32),
    ),
    scratch_types=[
        pltpu.VMEM((EPW,), jnp.int32),
        pltpu.VMEM((EPW,), jnp.int32),
        pltpu.VMEM((EPW,), jnp.int32),
        pltpu.VMEM((ECH,), jnp.int32),
        pltpu.VMEM((ECH,), jnp.int32),
        pltpu.VMEM((ECH,), jnp.float32),
        pltpu.VMEM((ECH,), jnp.float32),
        pltpu.VMEM((TAIL,), jnp.int32),
        pltpu.VMEM((TAIL,), jnp.float32),
        pltpu.VMEM((RPT,), jnp.float32),
        pltpu.VMEM_SHARED((NACC,), jnp.float32),
        pltpu.SemaphoreType.DMA,
        pltpu.SemaphoreType.DMA,
    ],
)
def _edge_prep(src_h, dst_h, degp_h, dstp_h,
               srcb, dstb, ndb, ic0, ic1, vc0, vc1, srct, valt, zb, deg_sh,
               s0, s1):
    cid = lax.axis_index("c")
    sid = lax.axis_index("s")
    wid = sid * NC + cid
    ebase = wid * EPW

    pltpu.async_copy(src_h.at[pl.ds(ebase, EPW)], srcb, s0)
    pltpu.async_copy(dst_h.at[pl.ds(ebase, EPW)], dstb, s1)
    for j in range(RPT // 16):
        zb[pl.ds(j * 16, 16)] = jnp.zeros((16,), jnp.float32)
    pltpu.sync_copy(zb, deg_sh.at[pl.ds(sid * RPT, RPT)])
    pltpu.make_async_copy(src_h.at[pl.ds(ebase, EPW)], srcb, s0).wait()
    pltpu.make_async_copy(dst_h.at[pl.ds(ebase, EPW)], dstb, s1).wait()
    plsc.subcore_barrier()

    bufs = ((ic0, vc0, s0), (ic1, vc1, s1))

    def compute(i, ic, vc):
        off = i * ECH
        for k in range(ECH // 16):
            sl = pl.ds(k * 16, 16)
            dsl = pl.ds(off + k * 16, 16)
            s16 = srcb[dsl]
            d16 = dstb[dsl]
            m = s16 == d16
            ndb[dsl] = jnp.where(m, TRASH, d16)
            ic[sl] = s16
            vc[sl] = jnp.where(m, 0.0, 1.0)

    for b in range(2):
        ic, vc, sem = bufs[b]
        compute(b, ic, vc)
        pltpu.async_copy(vc, deg_sh.at[ic], sem, add=True)

    def body(j, carry):
        for b in range(2):
            i = 2 * j + 2 + b
            ic, vc, sem = bufs[b]
            pltpu.make_async_copy(vc, deg_sh.at[ic], sem).wait()
            compute(i, ic, vc)
            pltpu.async_copy(vc, deg_sh.at[ic], sem, add=True)
        return carry

    lax.fori_loop(0, ENFULL // 2 - 1, body, 0)
    for b in range(2):
        ic, vc, sem = bufs[b]
        pltpu.make_async_copy(vc, deg_sh.at[ic], sem).wait()

    off = ENFULL * ECH
    dsl = pl.ds(off, 16)
    s16 = srcb[dsl]
    d16 = dstb[dsl]
    m = s16 == d16
    ndb[dsl] = jnp.where(m, TRASH, d16)
    srct[...] = s16
    valt[...] = jnp.where(m, 0.0, 1.0)
    pltpu.sync_copy(valt, deg_sh.at[srct], add=True)

    pltpu.sync_copy(ndb, dstp_h.at[pl.ds(ebase, EPW)])

    plsc.subcore_barrier()
    pltpu.sync_copy(
        deg_sh.at[pl.ds(sid * RPT, RPT)],
        degp_h.at[pl.ds(cid * NACC + sid * RPT, RPT)],
    )


@functools.partial(
    pl.kernel,
    mesh=_mesh,
    out_type=jax.ShapeDtypeStruct((NC * NACC, D), jnp.float32),
    scratch_types=[
        pltpu.VMEM((EPW,), jnp.int32),
        pltpu.VMEM((EPW,), jnp.int32),
        pltpu.VMEM((CH,), jnp.int32),
        pltpu.VMEM((CH,), jnp.int32),
        pltpu.VMEM((CH,), jnp.int32),
        pltpu.VMEM((CH,), jnp.int32),
        pltpu.VMEM((CH,), jnp.int32),
        pltpu.VMEM((CH,), jnp.int32),
        pltpu.VMEM((CH, D), jnp.float32),
        pltpu.VMEM((CH, D), jnp.float32),
        pltpu.VMEM((CH, D), jnp.float32),
        pltpu.VMEM((TAIL,), jnp.int32),
        pltpu.VMEM((TAIL,), jnp.int32),
        pltpu.VMEM((TAIL, D), jnp.float32),
        pltpu.VMEM((8, D), jnp.float32),
        pltpu.VMEM_SHARED((NACC, D), jnp.float32),
        pltpu.SemaphoreType.DMA,
        pltpu.SemaphoreType.DMA,
        pltpu.SemaphoreType.DMA,
        pltpu.SemaphoreType.DMA,
        pltpu.SemaphoreType.DMA,
        pltpu.SemaphoreType.DMA,
    ],
)
def _spmv(g_h, src_h, dstp_h, out_h,
          srcb, dstb, sc0, sc1, sc2, dc0, dc1, dc2, r0, r1, r2,
          srct, dstt, rowst, zb, acc_sh,
          sg0, sg1, sg2, ss0, ss1, ss2):
    cid = lax.axis_index("c")
    sid = lax.axis_index("s")
    wid = sid * NC + cid
    ebase = wid * EPW

    for r in range(8):
        for j in range(D // 16):
            zb[r, pl.ds(j * 16, 16)] = jnp.zeros((16,), jnp.float32)

    pltpu.async_copy(src_h.at[pl.ds(ebase, EPW)], srcb, sg0)
    pltpu.async_copy(dstp_h.at[pl.ds(ebase, EPW)], dstb, sg1)

    def zbody(i, carry):
        pltpu.sync_copy(zb, acc_sh.at[pl.ds(sid * RPT + i * 8, 8)])
        return carry

    lax.fori_loop(0, RPT // 8, zbody, 0)
    pltpu.make_async_copy(src_h.at[pl.ds(ebase, EPW)], srcb, sg0).wait()
    pltpu.make_async_copy(dstp_h.at[pl.ds(ebase, EPW)], dstb, sg1).wait()
    plsc.subcore_barrier()

    SC = (sc0, sc1, sc2)
    DC = (dc0, dc1, dc2)
    RW = (r0, r1, r2)
    SG = (sg0, sg1, sg2)
    SS = (ss0, ss1, ss2)

    def fill(i, sc, dc):
        off = i * CH
        for k in range(CH // 16):
            sl = pl.ds(k * 16, 16)
            sc[sl] = srcb[pl.ds(off + k * 16, 16)]
            dc[sl] = dstb[pl.ds(off + k * 16, 16)]

    def gather(i, b):
        fill(i, SC[b], DC[b])
        pltpu.async_copy(g_h.at[SC[b]], RW[b], SG[b])

    def wait_gather(b):
        pltpu.make_async_copy(g_h.at[SC[b]], RW[b], SG[b]).wait()

    def scatter(b):
        pltpu.async_copy(RW[b], acc_sh.at[DC[b]], SS[b], add=True)

    def wait_scatter(b):
        pltpu.make_async_copy(RW[b], acc_sh.at[DC[b]], SS[b]).wait()

    gather(0, 0)
    gather(1, 1)
    gather(2, 2)
    wait_gather(0)
    scatter(0)
    wait_scatter(0)
    gather(3, 0)
    wait_gather(1)
    scatter(1)
    wait_scatter(1)
    gather(4, 1)
    wait_gather(2)
    scatter(2)

    def body(j, carry):
        for b in range(3):
            i = 3 * j + b
            bt = (b + 2) % 3
            wait_scatter(bt)
            gather(i + 2, bt)
            wait_gather(b)
            scatter(b)
        return carry

    lax.fori_loop(1, NFULL // 3 - 1, body, 0)

    i = NFULL - 3
    wait_scatter(2)
    gather(i + 2, 2)
    wait_gather(0)
    scatter(0)
    wait_gather(1)
    scatter(1)
    wait_gather(2)
    scatter(2)
    wait_scatter(0)
    wait_scatter(1)
    wait_scatter(2)

    off = NFULL * CH
    for k in range(TAIL // 16):
        sl = pl.ds(k * 16, 16)
        srct[sl] = srcb[pl.ds(off + k * 16, 16)]
        dstt[sl] = dstb[pl.ds(off + k * 16, 16)]
    pltpu.async_copy(g_h.at[srct], rowst, sg0).wait()
    pltpu.sync_copy(rowst, acc_sh.at[dstt], add=True)

    plsc.subcore_barrier()
    pltpu.sync_copy(
        acc_sh.at[pl.ds(sid * RPT, RPT)],
        out_h.at[pl.ds(cid * NACC + sid * RPT, RPT)],
    )


BLK = 1000
GRID = N // BLK


def _prescale_body(degT_ref, x_ref, dinv_ref, g_ref):
    deg = degT_ref[...]
    d = deg[:, 0:1] + deg[:, 1:2]
    dinv = jnp.where(d > 0.0, lax.rsqrt(d), 0.0)
    dinv_ref[...] = dinv
    g_ref[...] = x_ref[...] * dinv


def _prescale(degT, x):
    return pl.pallas_call(
        _prescale_body,
        grid=(GRID,),
        in_specs=[
            pl.BlockSpec((BLK, NC), lambda i: (i, 0)),
            pl.BlockSpec((BLK, D), lambda i: (i, 0)),
        ],
        out_specs=[
            pl.BlockSpec((BLK, 1), lambda i: (i, 0)),
            pl.BlockSpec((BLK, D), lambda i: (i, 0)),
        ],
        out_shape=[
            jax.ShapeDtypeStruct((N, 1), jnp.float32),
            jax.ShapeDtypeStruct((N, D), jnp.float32),
        ],
    )(degT, x)


def _mid_body(dinv_ref, s1a_ref, s1b_ref, g2_ref):
    d = dinv_ref[...]
    g2_ref[...] = -(d * d) * (s1a_ref[...] + s1b_ref[...])


def _mid(dinv, sa, sb):
    return pl.pallas_call(
        _mid_body,
        grid=(GRID,),
        in_specs=[
            pl.BlockSpec((BLK, 1), lambda i: (i, 0)),
            pl.BlockSpec((BLK, D), lambda i: (i, 0)),
            pl.BlockSpec((BLK, D), lambda i: (i, 0)),
        ],
        out_specs=pl.BlockSpec((BLK, D), lambda i: (i, 0)),
        out_shape=jax.ShapeDtypeStruct((N, D), jnp.float32),
    )(dinv, sa, sb)


def _mm_body(x_ref, dinv_ref, s1a, s1b, s2a, s2b, w_ref, b_ref,
             out_ref, st_ref, acc):
    i = pl.program_id(0)
    d = dinv_ref[...]
    xv = x_ref[...]
    tx1 = -d * (s1a[...] + s1b[...])
    tx2 = -2.0 * d * (s2a[...] + s2b[...]) - xv
    o = (jnp.dot(xv, w_ref[0], preferred_element_type=jnp.float32)
         + jnp.dot(tx1, w_ref[1], preferred_element_type=jnp.float32)
         + jnp.dot(tx2, w_ref[2], preferred_element_type=jnp.float32)
         + b_ref[...])
    out_ref[...] = o

    @pl.when(i == 0)
    def _init():
        acc[...] = jnp.zeros_like(acc)

    acc[0:1, :] += jnp.sum(o, axis=0, keepdims=True)
    acc[1:2, :] += jnp.sum(o * o, axis=0, keepdims=True)

    @pl.when(i == GRID - 1)
    def _fin():
        st_ref[...] = acc[...]


def _mm(x, dinv, s1a, s1b, s2a, s2b, W, b2):
    return pl.pallas_call(
        _mm_body,
        grid=(GRID,),
        in_specs=[
            pl.BlockSpec((BLK, D), lambda i: (i, 0)),
            pl.BlockSpec((BLK, 1), lambda i: (i, 0)),
            pl.BlockSpec((BLK, D), lambda i: (i, 0)),
            pl.BlockSpec((BLK, D), lambda i: (i, 0)),
            pl.BlockSpec((BLK, D), lambda i: (i, 0)),
            pl.BlockSpec((BLK, D), lambda i: (i, 0)),
            pl.BlockSpec((3, D, D), lambda i: (0, 0, 0)),
            pl.BlockSpec((1, D), lambda i: (0, 0)),
        ],
        out_specs=[
            pl.BlockSpec((BLK, D), lambda i: (i, 0)),
            pl.BlockSpec((2, D), lambda i: (0, 0)),
        ],
        out_shape=[
            jax.ShapeDtypeStruct((N, D), jnp.float32),
            jax.ShapeDtypeStruct((2, D), jnp.float32),
        ],
        scratch_shapes=[pltpu.VMEM((2, D), jnp.float32)],
    )(x, dinv, s1a, s1b, s2a, s2b, W, b2)


def _norm_body(o_ref, st_ref, gam_ref, bet_ref, y_ref):
    st = st_ref[...]
    mean = st[0:1, :] * (1.0 / N)
    var = st[1:2, :] * (1.0 / N) - mean * mean
    scale = lax.rsqrt(var + EPS) * gam_ref[...]
    y = (o_ref[...] - mean) * scale + bet_ref[...]
    y_ref[...] = jnp.where(y >= 0.0, y, ALPHA * y)


def _norm(o, st, gam2, bet2):
    return pl.pallas_call(
        _norm_body,
        grid=(GRID,),
        in_specs=[
            pl.BlockSpec((BLK, D), lambda i: (i, 0)),
            pl.BlockSpec((2, D), lambda i: (0, 0)),
            pl.BlockSpec((1, D), lambda i: (0, 0)),
            pl.BlockSpec((1, D), lambda i: (0, 0)),
        ],
        out_specs=pl.BlockSpec((BLK, D), lambda i: (i, 0)),
        out_shape=jax.ShapeDtypeStruct((N, D), jnp.float32),
    )(o, st, gam2, bet2)


@jax.jit
def kernel(x, edge_idx, W, b, gamma, beta):
    src = edge_idx[0]
    dst = edge_idx[1]
    degp, dstp = _edge_prep(src, dst)
    degT = jnp.transpose(degp.reshape(NC, NACC)[:, :N])
    dinv, g1 = _prescale(degT, x)
    s1 = _spmv(g1, src, dstp)
    s1a, s1b = s1[:N], s1[NACC:NACC + N]
    g2 = _mid(dinv, s1a, s1b)
    s2 = _spmv(g2, src, dstp)
    s2a, s2b = s2[:N], s2[NACC:NACC + N]
    outp, stats = _mm(x, dinv, s1a, s1b, s2a, s2b, W, b.reshape(1, D))
    return _norm(outp, stats, gamma.reshape(1, D), beta.reshape(1, D))


# speedup vs baseline: 24.8913x; 1.0147x over previous
import functools

import jax
import jax.numpy as jnp
from jax import lax
from jax.experimental import pallas as pl
from jax.experimental.pallas import tpu as pltpu
from jax.experimental.pallas import tpu_sc as plsc

N = 10000
E = 320000
D = 128
EPS = 1e-5
ALPHA = 0.01

NC = 2
NS = 16
NW = NC * NS
EPW = E // NW
ECH = 128
ENFULL = EPW // ECH
CH = 64
NFULL = EPW // CH
TAIL = EPW - NFULL * CH
TRASH = N
RPT = 640
NACC = NS * RPT

_mesh = plsc.VectorSubcoreMesh(core_axis_name="c", subcore_axis_name="s")


@functools.partial(
    pl.kernel,
    mesh=_mesh,
    out_type=(
        jax.ShapeDtypeStruct((NC * NACC,), jnp.float32),
        jax.ShapeDtypeStruct((E,), jnp.int32),
    ),
    scratch_types=[
        pltpu.VMEM((EPW,), jnp.int32),
        pltpu.VMEM((EPW,), jnp.int32),
        pltpu.VMEM((EPW,), jnp.int32),
        pltpu.VMEM((ECH,), jnp.int32),
        pltpu.VMEM((ECH,), jnp.int32),
        pltpu.VMEM((ECH,), jnp.float32),
        pltpu.VMEM((ECH,), jnp.float32),
        pltpu.VMEM((TAIL,), jnp.int32),
        pltpu.VMEM((TAIL,), jnp.float32),
        pltpu.VMEM((RPT,), jnp.float32),
        pltpu.VMEM_SHARED((NACC,), jnp.float32),
        pltpu.SemaphoreType.DMA,
        pltpu.SemaphoreType.DMA,
    ],
)
def _edge_prep(src_h, dst_h, degp_h, dstp_h,
               srcb, dstb, ndb, ic0, ic1, vc0, vc1, srct, valt, zb, deg_sh,
               s0, s1):
    cid = lax.axis_index("c")
    sid = lax.axis_index("s")
    wid = sid * NC + cid
    ebase = wid * EPW

    pltpu.async_copy(src_h.at[pl.ds(ebase, EPW)], srcb, s0)
    pltpu.async_copy(dst_h.at[pl.ds(ebase, EPW)], dstb, s1)
    for j in range(RPT // 16):
        zb[pl.ds(j * 16, 16)] = jnp.zeros((16,), jnp.float32)
    pltpu.sync_copy(zb, deg_sh.at[pl.ds(sid * RPT, RPT)])
    pltpu.make_async_copy(src_h.at[pl.ds(ebase, EPW)], srcb, s0).wait()
    pltpu.make_async_copy(dst_h.at[pl.ds(ebase, EPW)], dstb, s1).wait()
    plsc.subcore_barrier()

    bufs = ((ic0, vc0, s0), (ic1, vc1, s1))

    def compute(i, ic, vc):
        off = i * ECH
        for k in range(ECH // 16):
            sl = pl.ds(k * 16, 16)
            dsl = pl.ds(off + k * 16, 16)
            s16 = srcb[dsl]
            d16 = dstb[dsl]
            m = s16 == d16
            ndb[dsl] = jnp.where(m, TRASH, d16)
            ic[sl] = s16
            vc[sl] = jnp.where(m, 0.0, 1.0)

    for b in range(2):
        ic, vc, sem = bufs[b]
        compute(b, ic, vc)
        pltpu.async_copy(vc, deg_sh.at[ic], sem, add=True)

    def body(j, carry):
        for b in range(2):
            i = 2 * j + 2 + b
            ic, vc, sem = bufs[b]
            pltpu.make_async_copy(vc, deg_sh.at[ic], sem).wait()
            compute(i, ic, vc)
            pltpu.async_copy(vc, deg_sh.at[ic], sem, add=True)
        return carry

    lax.fori_loop(0, ENFULL // 2 - 1, body, 0)
    for b in range(2):
        ic, vc, sem = bufs[b]
        pltpu.make_async_copy(vc, deg_sh.at[ic], sem).wait()

    off = ENFULL * ECH
    dsl = pl.ds(off, 16)
    s16 = srcb[dsl]
    d16 = dstb[dsl]
    m = s16 == d16
    ndb[dsl] = jnp.where(m, TRASH, d16)
    srct[...] = s16
    valt[...] = jnp.where(m, 0.0, 1.0)
    pltpu.sync_copy(valt, deg_sh.at[srct], add=True)

    pltpu.sync_copy(ndb, dstp_h.at[pl.ds(ebase, EPW)])

    plsc.subcore_barrier()
    pltpu.sync_copy(
        deg_sh.at[pl.ds(sid * RPT, RPT)],
        degp_h.at[pl.ds(cid * NACC + sid * RPT, RPT)],
    )


@functools.partial(
    pl.kernel,
    mesh=_mesh,
    out_type=jax.ShapeDtypeStruct((NC * NACC, D), jnp.float32),
    scratch_types=[
        pltpu.VMEM((EPW,), jnp.int32),
        pltpu.VMEM((EPW,), jnp.int32),
        pltpu.VMEM((CH,), jnp.int32),
        pltpu.VMEM((CH,), jnp.int32),
        pltpu.VMEM((CH,), jnp.int32),
        pltpu.VMEM((CH,), jnp.int32),
        pltpu.VMEM((CH,), jnp.int32),
        pltpu.VMEM((CH,), jnp.int32),
        pltpu.VMEM((CH, D), jnp.float32),
        pltpu.VMEM((CH, D), jnp.float32),
        pltpu.VMEM((CH, D), jnp.float32),
        pltpu.VMEM((TAIL,), jnp.int32),
        pltpu.VMEM((TAIL,), jnp.int32),
        pltpu.VMEM((TAIL, D), jnp.float32),
        pltpu.VMEM((8, D), jnp.float32),
        pltpu.VMEM_SHARED((NACC, D), jnp.float32),
        pltpu.SemaphoreType.DMA,
        pltpu.SemaphoreType.DMA,
        pltpu.SemaphoreType.DMA,
        pltpu.SemaphoreType.DMA,
        pltpu.SemaphoreType.DMA,
        pltpu.SemaphoreType.DMA,
    ],
)
def _spmv(g_h, src_h, dstp_h, out_h,
          srcb, dstb, sc0, sc1, sc2, dc0, dc1, dc2, r0, r1, r2,
          srct, dstt, rowst, zb, acc_sh,
          sg0, sg1, sg2, ss0, ss1, ss2):
    cid = lax.axis_index("c")
    sid = lax.axis_index("s")
    wid = sid * NC + cid
    ebase = wid * EPW

    for r in range(8):
        for j in range(D // 16):
            zb[r, pl.ds(j * 16, 16)] = jnp.zeros((16,), jnp.float32)

    pltpu.async_copy(src_h.at[pl.ds(ebase, EPW)], srcb, sg0)
    pltpu.async_copy(dstp_h.at[pl.ds(ebase, EPW)], dstb, sg1)

    def zbody(i, carry):
        pltpu.sync_copy(zb, acc_sh.at[pl.ds(sid * RPT + i * 8, 8)])
        return carry

    lax.fori_loop(0, RPT // 8, zbody, 0)
    pltpu.make_async_copy(src_h.at[pl.ds(ebase, EPW)], srcb, sg0).wait()
    pltpu.make_async_copy(dstp_h.at[pl.ds(ebase, EPW)], dstb, sg1).wait()
    plsc.subcore_barrier()

    SC = (sc0, sc1, sc2)
    DC = (dc0, dc1, dc2)
    RW = (r0, r1, r2)
    SG = (sg0, sg1, sg2)
    SS = (ss0, ss1, ss2)

    def fill(i, sc, dc):
        off = i * CH
        for k in range(CH // 16):
            sl = pl.ds(k * 16, 16)
            sc[sl] = srcb[pl.ds(off + k * 16, 16)]
            dc[sl] = dstb[pl.ds(off + k * 16, 16)]

    def gather(i, b):
        fill(i, SC[b], DC[b])
        pltpu.async_copy(g_h.at[SC[b]], RW[b], SG[b])

    def wait_gather(b):
        pltpu.make_async_copy(g_h.at[SC[b]], RW[b], SG[b]).wait()

    def scatter(b):
        pltpu.async_copy(RW[b], acc_sh.at[DC[b]], SS[b], add=True)

    def wait_scatter(b):
        pltpu.make_async_copy(RW[b], acc_sh.at[DC[b]], SS[b]).wait()

    gather(0, 0)
    gather(1, 1)
    gather(2, 2)
    wait_gather(0)
    scatter(0)
    wait_scatter(0)
    gather(3, 0)
    wait_gather(1)
    scatter(1)
    wait_scatter(1)
    gather(4, 1)
    wait_gather(2)
    scatter(2)

    def body(j, carry):
        for b in range(3):
            i = 3 * j + b
            bt = (b + 2) % 3
            wait_scatter(bt)
            gather(i + 2, bt)
            wait_gather(b)
            scatter(b)
        return carry

    lax.fori_loop(1, NFULL // 3 - 1, body, 0)

    i = NFULL - 3
    wait_scatter(2)
    gather(i + 2, 2)
    wait_gather(0)
    scatter(0)
    wait_gather(1)
    scatter(1)
    wait_gather(2)
    scatter(2)
    wait_scatter(0)
    wait_scatter(1)
    wait_scatter(2)

    off = NFULL * CH
    for k in range(TAIL // 16):
        sl = pl.ds(k * 16, 16)
        srct[sl] = srcb[pl.ds(off + k * 16, 16)]
        dstt[sl] = dstb[pl.ds(off + k * 16, 16)]
    pltpu.async_copy(g_h.at[srct], rowst, sg0).wait()
    pltpu.sync_copy(rowst, acc_sh.at[dstt], add=True)

    plsc.subcore_barrier()
    pltpu.sync_copy(
        acc_sh.at[pl.ds(sid * RPT, RPT)],
        out_h.at[pl.ds(cid * NACC + sid * RPT, RPT)],
    )


BLK = 1000
GRID = N // BLK


def _prescale_body(degT_ref, x_ref, dinv_ref, g_ref):
    deg = degT_ref[...]
    d = deg[:, 0:1] + deg[:, 1:2]
    dinv = jnp.where(d > 0.0, lax.rsqrt(d), 0.0)
    dinv_ref[...] = dinv
    g_ref[...] = x_ref[...] * dinv


def _prescale(degT, x):
    return pl.pallas_call(
        _prescale_body,
        grid=(GRID,),
        in_specs=[
            pl.BlockSpec((BLK, NC), lambda i: (i, 0)),
            pl.BlockSpec((BLK, D), lambda i: (i, 0)),
        ],
        out_specs=[
            pl.BlockSpec((BLK, 1), lambda i: (i, 0)),
            pl.BlockSpec((BLK, D), lambda i: (i, 0)),
        ],
        out_shape=[
            jax.ShapeDtypeStruct((N, 1), jnp.float32),
            jax.ShapeDtypeStruct((N, D), jnp.float32),
        ],
    )(degT, x)


def _mid_body(dinv_ref, s1a_ref, s1b_ref, g2_ref):
    d = dinv_ref[...]
    g2_ref[...] = -(d * d) * (s1a_ref[...] + s1b_ref[...])


def _mid(dinv, sa, sb):
    return pl.pallas_call(
        _mid_body,
        grid=(GRID,),
        in_specs=[
            pl.BlockSpec((BLK, 1), lambda i: (i, 0)),
            pl.BlockSpec((BLK, D), lambda i: (i, 0)),
            pl.BlockSpec((BLK, D), lambda i: (i, 0)),
        ],
        out_specs=pl.BlockSpec((BLK, D), lambda i: (i, 0)),
        out_shape=jax.ShapeDtypeStruct((N, D), jnp.float32),
    )(dinv, sa, sb)


def _mmn_body(x_ref, dinv_ref, s1a, s1b, s2a, s2b, w_ref, b_ref,
              gam_ref, bet_ref, y_ref, acc, oall):
    p = pl.program_id(0)
    i = pl.program_id(1)

    @pl.when(p == 0)
    def _compute():
        d = dinv_ref[...]
        xv = x_ref[...]
        tx1 = -d * (s1a[...] + s1b[...])
        tx2 = -2.0 * d * (s2a[...] + s2b[...]) - xv
        o = (jnp.dot(xv, w_ref[0], preferred_element_type=jnp.float32)
             + jnp.dot(tx1, w_ref[1], preferred_element_type=jnp.float32)
             + jnp.dot(tx2, w_ref[2], preferred_element_type=jnp.float32)
             + b_ref[...])
        oall[pl.ds(i * BLK, BLK), :] = o

        @pl.when(i == 0)
        def _init():
            acc[...] = jnp.zeros_like(acc)

        acc[0:1, :] += jnp.sum(o, axis=0, keepdims=True)
        acc[1:2, :] += jnp.sum(o * o, axis=0, keepdims=True)

    @pl.when(p == 1)
    def _normalize():
        st = acc[...]
        mean = st[0:1, :] * (1.0 / N)
        var = st[1:2, :] * (1.0 / N) - mean * mean
        scale = lax.rsqrt(var + EPS) * gam_ref[...]
        o = oall[pl.ds(i * BLK, BLK), :]
        y = (o - mean) * scale + bet_ref[...]
        y_ref[...] = jnp.where(y >= 0.0, y, ALPHA * y)


def _mmn(x, dinv, s1a, s1b, s2a, s2b, W, b2, gam2, bet2):
    blk = lambda p, i: (i * (1 - p), 0)
    return pl.pallas_call(
        _mmn_body,
        grid=(2, GRID),
        in_specs=[
            pl.BlockSpec((BLK, D), blk),
            pl.BlockSpec((BLK, 1), blk),
            pl.BlockSpec((BLK, D), blk),
            pl.BlockSpec((BLK, D), blk),
            pl.BlockSpec((BLK, D), blk),
            pl.BlockSpec((BLK, D), blk),
            pl.BlockSpec((3, D, D), lambda p, i: (0, 0, 0)),
            pl.BlockSpec((1, D), lambda p, i: (0, 0)),
            pl.BlockSpec((1, D), lambda p, i: (0, 0)),
            pl.BlockSpec((1, D), lambda p, i: (0, 0)),
        ],
        out_specs=pl.BlockSpec((BLK, D), lambda p, i: (i * p, 0)),
        out_shape=jax.ShapeDtypeStruct((N, D), jnp.float32),
        scratch_shapes=[
            pltpu.VMEM((2, D), jnp.float32),
            pltpu.VMEM((N, D), jnp.float32),
        ],
    )(x, dinv, s1a, s1b, s2a, s2b, W, b2, gam2, bet2)


@jax.jit
def kernel(x, edge_idx, W, b, gamma, beta):
    src = edge_idx[0]
    dst = edge_idx[1]
    degp, dstp = _edge_prep(src, dst)
    degT = jnp.transpose(degp.reshape(NC, NACC)[:, :N])
    dinv, g1 = _prescale(degT, x)
    s1 = _spmv(g1, src, dstp)
    s1a, s1b = s1[:N], s1[NACC:NACC + N]
    g2 = _mid(dinv, s1a, s1b)
    s2 = _spmv(g2, src, dstp)
    s2a, s2b = s2[:N], s2[NACC:NACC + N]
    return _mmn(x, dinv, s1a, s1b, s2a, s2b, W, b.reshape(1, D),
                gamma.reshape(1, D), beta.reshape(1, D))


# speedup vs baseline: 25.2168x; 1.0131x over previous
import functools

import jax
import jax.numpy as jnp
from jax import lax
from jax.experimental import pallas as pl
from jax.experimental.pallas import tpu as pltpu
from jax.experimental.pallas import tpu_sc as plsc

N = 10000
E = 320000
D = 128
EPS = 1e-5
ALPHA = 0.01

NC = 2
NS = 16
NW = NC * NS
EPW = E // NW
ECH = 128
ENFULL = EPW // ECH
CH = 96
NFULL = EPW // CH
TAIL = EPW - NFULL * CH
TRASH = N
RPT = 640
NACC = NS * RPT

_mesh = plsc.VectorSubcoreMesh(core_axis_name="c", subcore_axis_name="s")


@functools.partial(
    pl.kernel,
    mesh=_mesh,
    out_type=(
        jax.ShapeDtypeStruct((NC * NACC,), jnp.float32),
        jax.ShapeDtypeStruct((E,), jnp.int32),
    ),
    scratch_types=[
        pltpu.VMEM((EPW,), jnp.int32),
        pltpu.VMEM((EPW,), jnp.int32),
        pltpu.VMEM((EPW,), jnp.int32),
        pltpu.VMEM((ECH,), jnp.int32),
        pltpu.VMEM((ECH,), jnp.int32),
        pltpu.VMEM((ECH,), jnp.float32),
        pltpu.VMEM((ECH,), jnp.float32),
        pltpu.VMEM((TAIL,), jnp.int32),
        pltpu.VMEM((TAIL,), jnp.float32),
        pltpu.VMEM((RPT,), jnp.float32),
        pltpu.VMEM_SHARED((NACC,), jnp.float32),
        pltpu.SemaphoreType.DMA,
        pltpu.SemaphoreType.DMA,
    ],
)
def _edge_prep(src_h, dst_h, degp_h, dstp_h,
               srcb, dstb, ndb, ic0, ic1, vc0, vc1, srct, valt, zb, deg_sh,
               s0, s1):
    cid = lax.axis_index("c")
    sid = lax.axis_index("s")
    wid = sid * NC + cid
    ebase = wid * EPW

    pltpu.async_copy(src_h.at[pl.ds(ebase, EPW)], srcb, s0)
    pltpu.async_copy(dst_h.at[pl.ds(ebase, EPW)], dstb, s1)
    for j in range(RPT // 16):
        zb[pl.ds(j * 16, 16)] = jnp.zeros((16,), jnp.float32)
    pltpu.sync_copy(zb, deg_sh.at[pl.ds(sid * RPT, RPT)])
    pltpu.make_async_copy(src_h.at[pl.ds(ebase, EPW)], srcb, s0).wait()
    pltpu.make_async_copy(dst_h.at[pl.ds(ebase, EPW)], dstb, s1).wait()
    plsc.subcore_barrier()

    bufs = ((ic0, vc0, s0), (ic1, vc1, s1))

    def compute(i, ic, vc):
        off = i * ECH
        for k in range(ECH // 16):
            sl = pl.ds(k * 16, 16)
            dsl = pl.ds(off + k * 16, 16)
            s16 = srcb[dsl]
            d16 = dstb[dsl]
            m = s16 == d16
            ndb[dsl] = jnp.where(m, TRASH, d16)
            ic[sl] = s16
            vc[sl] = jnp.where(m, 0.0, 1.0)

    for b in range(2):
        ic, vc, sem = bufs[b]
        compute(b, ic, vc)
        pltpu.async_copy(vc, deg_sh.at[ic], sem, add=True)

    def body(j, carry):
        for b in range(2):
            i = 2 * j + 2 + b
            ic, vc, sem = bufs[b]
            pltpu.make_async_copy(vc, deg_sh.at[ic], sem).wait()
            compute(i, ic, vc)
            pltpu.async_copy(vc, deg_sh.at[ic], sem, add=True)
        return carry

    lax.fori_loop(0, ENFULL // 2 - 1, body, 0)
    for b in range(2):
        ic, vc, sem = bufs[b]
        pltpu.make_async_copy(vc, deg_sh.at[ic], sem).wait()

    off = ENFULL * ECH
    dsl = pl.ds(off, 16)
    s16 = srcb[dsl]
    d16 = dstb[dsl]
    m = s16 == d16
    ndb[dsl] = jnp.where(m, TRASH, d16)
    srct[...] = s16
    valt[...] = jnp.where(m, 0.0, 1.0)
    pltpu.sync_copy(valt, deg_sh.at[srct], add=True)

    pltpu.sync_copy(ndb, dstp_h.at[pl.ds(ebase, EPW)])

    plsc.subcore_barrier()
    pltpu.sync_copy(
        deg_sh.at[pl.ds(sid * RPT, RPT)],
        degp_h.at[pl.ds(cid * NACC + sid * RPT, RPT)],
    )


@functools.partial(
    pl.kernel,
    mesh=_mesh,
    out_type=jax.ShapeDtypeStruct((NC * NACC, D), jnp.float32),
    scratch_types=[
        pltpu.VMEM((EPW,), jnp.int32),
        pltpu.VMEM((CH,), jnp.int32),
        pltpu.VMEM((CH,), jnp.int32),
        pltpu.VMEM((CH,), jnp.int32),
        pltpu.VMEM((CH, D), jnp.float32),
        pltpu.VMEM((CH, D), jnp.float32),
        pltpu.VMEM((CH, D), jnp.float32),
        pltpu.VMEM((TAIL,), jnp.int32),
        pltpu.VMEM((4, D), jnp.float32),
        pltpu.VMEM_SHARED((NACC, D), jnp.float32),
        pltpu.SemaphoreType.DMA,
        pltpu.SemaphoreType.DMA,
        pltpu.SemaphoreType.DMA,
        pltpu.SemaphoreType.DMA,
        pltpu.SemaphoreType.DMA,
        pltpu.SemaphoreType.DMA,
        pltpu.SemaphoreType.DMA,
        pltpu.SemaphoreType.DMA,
        pltpu.SemaphoreType.DMA,
    ],
)
def _spmv(g_h, src_h, dstp_h, out_h,
          srcb, dc0, dc1, dc2, r0, r1, r2, dstt, zb, acc_sh,
          sg0, sg1, sg2, sd0, sd1, sd2, ss0, ss1, ss2):
    cid = lax.axis_index("c")
    sid = lax.axis_index("s")
    wid = sid * NC + cid
    ebase = wid * EPW

    for r in range(4):
        for j in range(D // 16):
            zb[r, pl.ds(j * 16, 16)] = jnp.zeros((16,), jnp.float32)

    pltpu.async_copy(src_h.at[pl.ds(ebase, EPW)], srcb, sg0)

    def zbody(i, carry):
        pltpu.sync_copy(zb, acc_sh.at[pl.ds(sid * RPT + i * 4, 4)])
        return carry

    lax.fori_loop(0, RPT // 4, zbody, 0)
    pltpu.make_async_copy(src_h.at[pl.ds(ebase, EPW)], srcb, sg0).wait()
    plsc.subcore_barrier()

    DC = (dc0, dc1, dc2)
    RW = (r0, r1, r2)
    SG = (sg0, sg1, sg2)
    SD = (sd0, sd1, sd2)
    SS = (ss0, ss1, ss2)

    def gather(i, b):
        off = i * CH
        pltpu.async_copy(dstp_h.at[pl.ds(ebase + off, CH)], DC[b], SD[b])
        pltpu.async_copy(g_h.at[srcb.at[pl.ds(off, CH)]], RW[b], SG[b])

    def wait_gather(i, b):
        off = i * CH
        pltpu.make_async_copy(dstp_h.at[pl.ds(ebase + off, CH)], DC[b],
                              SD[b]).wait()
        pltpu.make_async_copy(g_h.at[srcb.at[pl.ds(off, CH)]], RW[b],
                              SG[b]).wait()

    def scatter(b):
        pltpu.async_copy(RW[b], acc_sh.at[DC[b]], SS[b], add=True)

    def wait_scatter(b):
        pltpu.make_async_copy(RW[b], acc_sh.at[DC[b]], SS[b]).wait()

    gather(0, 0)
    gather(1, 1)
    gather(2, 2)
    wait_gather(0, 0)
    scatter(0)
    wait_scatter(0)
    gather(3, 0)
    wait_gather(1, 1)
    scatter(1)
    wait_scatter(1)
    gather(4, 1)
    wait_gather(2, 2)
    scatter(2)

    def body(j, carry):
        for b in range(3):
            i = 3 * j + b
            bt = (b + 2) % 3
            wait_scatter(bt)
            gather(i + 2, bt)
            wait_gather(i, b)
            scatter(b)
        return carry

    lax.fori_loop(1, (NFULL - 2) // 3, body, 0)

    wait_scatter(2)
    wait_gather(NFULL - 2, 0)
    scatter(0)
    wait_scatter(0)
    wait_gather(NFULL - 1, 1)
    scatter(1)
    wait_scatter(1)

    off = NFULL * CH
    pltpu.async_copy(dstp_h.at[pl.ds(ebase + off, TAIL)], dstt, sd0).wait()
    pltpu.async_copy(g_h.at[srcb.at[pl.ds(off, TAIL)]],
                     r0.at[pl.ds(0, TAIL)], sg0).wait()
    pltpu.sync_copy(r0.at[pl.ds(0, TAIL)], acc_sh.at[dstt], add=True)

    plsc.subcore_barrier()
    pltpu.sync_copy(
        acc_sh.at[pl.ds(sid * RPT, RPT)],
        out_h.at[pl.ds(cid * NACC + sid * RPT, RPT)],
    )


BLK = 1000
GRID = N // BLK


def _prescale_body(degT_ref, x_ref, dinv_ref, g_ref):
    deg = degT_ref[...]
    d = deg[:, 0:1] + deg[:, 1:2]
    dinv = jnp.where(d > 0.0, lax.rsqrt(d), 0.0)
    dinv_ref[...] = dinv
    g_ref[...] = x_ref[...] * dinv


def _prescale(degT, x):
    return pl.pallas_call(
        _prescale_body,
        grid=(GRID,),
        in_specs=[
            pl.BlockSpec((BLK, NC), lambda i: (i, 0)),
            pl.BlockSpec((BLK, D), lambda i: (i, 0)),
        ],
        out_specs=[
            pl.BlockSpec((BLK, 1), lambda i: (i, 0)),
            pl.BlockSpec((BLK, D), lambda i: (i, 0)),
        ],
        out_shape=[
            jax.ShapeDtypeStruct((N, 1), jnp.float32),
            jax.ShapeDtypeStruct((N, D), jnp.float32),
        ],
    )(degT, x)


def _mid_body(dinv_ref, s1a_ref, s1b_ref, g2_ref):
    d = dinv_ref[...]
    g2_ref[...] = -(d * d) * (s1a_ref[...] + s1b_ref[...])


def _mid(dinv, sa, sb):
    return pl.pallas_call(
        _mid_body,
        grid=(GRID,),
        in_specs=[
            pl.BlockSpec((BLK, 1), lambda i: (i, 0)),
            pl.BlockSpec((BLK, D), lambda i: (i, 0)),
            pl.BlockSpec((BLK, D), lambda i: (i, 0)),
        ],
        out_specs=pl.BlockSpec((BLK, D), lambda i: (i, 0)),
        out_shape=jax.ShapeDtypeStruct((N, D), jnp.float32),
    )(dinv, sa, sb)


def _mmn_body(x_ref, dinv_ref, s1a, s1b, s2a, s2b, w_ref, b_ref,
              gam_ref, bet_ref, y_ref, acc, oall):
    p = pl.program_id(0)
    i = pl.program_id(1)

    @pl.when(p == 0)
    def _compute():
        d = dinv_ref[...]
        xv = x_ref[...]
        tx1 = -d * (s1a[...] + s1b[...])
        tx2 = -2.0 * d * (s2a[...] + s2b[...]) - xv
        o = (jnp.dot(xv, w_ref[0], preferred_element_type=jnp.float32)
             + jnp.dot(tx1, w_ref[1], preferred_element_type=jnp.float32)
             + jnp.dot(tx2, w_ref[2], preferred_element_type=jnp.float32)
             + b_ref[...])
        oall[pl.ds(i * BLK, BLK), :] = o

        @pl.when(i == 0)
        def _init():
            acc[...] = jnp.zeros_like(acc)

        acc[0:1, :] += jnp.sum(o, axis=0, keepdims=True)
        acc[1:2, :] += jnp.sum(o * o, axis=0, keepdims=True)

    @pl.when(p == 1)
    def _normalize():
        st = acc[...]
        mean = st[0:1, :] * (1.0 / N)
        var = st[1:2, :] * (1.0 / N) - mean * mean
        scale = lax.rsqrt(var + EPS) * gam_ref[...]
        o = oall[pl.ds(i * BLK, BLK), :]
        y = (o - mean) * scale + bet_ref[...]
        y_ref[...] = jnp.where(y >= 0.0, y, ALPHA * y)


def _mmn(x, dinv, s1a, s1b, s2a, s2b, W, b2, gam2, bet2):
    blk = lambda p, i: (i * (1 - p), 0)
    return pl.pallas_call(
        _mmn_body,
        grid=(2, GRID),
        in_specs=[
            pl.BlockSpec((BLK, D), blk),
            pl.BlockSpec((BLK, 1), blk),
            pl.BlockSpec((BLK, D), blk),
            pl.BlockSpec((BLK, D), blk),
            pl.BlockSpec((BLK, D), blk),
            pl.BlockSpec((BLK, D), blk),
            pl.BlockSpec((3, D, D), lambda p, i: (0, 0, 0)),
            pl.BlockSpec((1, D), lambda p, i: (0, 0)),
            pl.BlockSpec((1, D), lambda p, i: (0, 0)),
            pl.BlockSpec((1, D), lambda p, i: (0, 0)),
        ],
        out_specs=pl.BlockSpec((BLK, D), lambda p, i: (i * p, 0)),
        out_shape=jax.ShapeDtypeStruct((N, D), jnp.float32),
        scratch_shapes=[
            pltpu.VMEM((2, D), jnp.float32),
            pltpu.VMEM((N, D), jnp.float32),
        ],
    )(x, dinv, s1a, s1b, s2a, s2b, W, b2, gam2, bet2)


@jax.jit
def kernel(x, edge_idx, W, b, gamma, beta):
    src = edge_idx[0]
    dst = edge_idx[1]
    degp, dstp = _edge_prep(src, dst)
    degT = jnp.transpose(degp.reshape(NC, NACC)[:, :N])
    dinv, g1 = _prescale(degT, x)
    s1 = _spmv(g1, src, dstp)
    s1a, s1b = s1[:N], s1[NACC:NACC + N]
    g2 = _mid(dinv, s1a, s1b)
    s2 = _spmv(g2, src, dstp)
    s2a, s2b = s2[:N], s2[NACC:NACC + N]
    return _mmn(x, dinv, s1a, s1b, s2a, s2b, W, b.reshape(1, D),
                gamma.reshape(1, D), beta.reshape(1, D))


# speedup vs baseline: 25.4322x; 1.0085x over previous
import functools

import jax
import jax.numpy as jnp
from jax import lax
from jax.experimental import pallas as pl
from jax.experimental.pallas import tpu as pltpu
from jax.experimental.pallas import tpu_sc as plsc

N = 10000
E = 320000
D = 128
EPS = 1e-5
ALPHA = 0.01

NC = 2
NS = 16
NW = NC * NS
EPW = E // NW
ECH = 128
ENFULL = EPW // ECH
CH = 96
NFULL = EPW // CH
TAIL = EPW - NFULL * CH
TRASH = N
RPT = 640
NACC = NS * RPT

_mesh = plsc.VectorSubcoreMesh(core_axis_name="c", subcore_axis_name="s")


@functools.partial(
    pl.kernel,
    mesh=_mesh,
    out_type=(
        jax.ShapeDtypeStruct((NC * NACC,), jnp.float32),
        jax.ShapeDtypeStruct((E,), jnp.int32),
    ),
    scratch_types=[
        pltpu.VMEM((EPW,), jnp.int32),
        pltpu.VMEM((EPW,), jnp.int32),
        pltpu.VMEM((EPW,), jnp.int32),
        pltpu.VMEM((ECH,), jnp.int32),
        pltpu.VMEM((ECH,), jnp.int32),
        pltpu.VMEM((ECH,), jnp.float32),
        pltpu.VMEM((ECH,), jnp.float32),
        pltpu.VMEM((TAIL,), jnp.int32),
        pltpu.VMEM((TAIL,), jnp.float32),
        pltpu.VMEM((RPT,), jnp.float32),
        pltpu.VMEM_SHARED((NACC,), jnp.float32),
        pltpu.SemaphoreType.DMA,
        pltpu.SemaphoreType.DMA,
    ],
)
def _edge_prep(src_h, dst_h, degp_h, dstp_h,
               srcb, dstb, ndb, ic0, ic1, vc0, vc1, srct, valt, zb, deg_sh,
               s0, s1):
    cid = lax.axis_index("c")
    sid = lax.axis_index("s")
    wid = sid * NC + cid
    ebase = wid * EPW

    pltpu.async_copy(src_h.at[pl.ds(ebase, EPW)], srcb, s0)
    pltpu.async_copy(dst_h.at[pl.ds(ebase, EPW)], dstb, s1)
    for j in range(RPT // 16):
        zb[pl.ds(j * 16, 16)] = jnp.zeros((16,), jnp.float32)
    pltpu.sync_copy(zb, deg_sh.at[pl.ds(sid * RPT, RPT)])
    pltpu.make_async_copy(src_h.at[pl.ds(ebase, EPW)], srcb, s0).wait()
    pltpu.make_async_copy(dst_h.at[pl.ds(ebase, EPW)], dstb, s1).wait()
    plsc.subcore_barrier()

    bufs = ((ic0, vc0, s0), (ic1, vc1, s1))

    def compute(i, ic, vc):
        off = i * ECH
        for k in range(ECH // 16):
            sl = pl.ds(k * 16, 16)
            dsl = pl.ds(off + k * 16, 16)
            s16 = srcb[dsl]
            d16 = dstb[dsl]
            m = s16 == d16
            ndb[dsl] = jnp.where(m, TRASH, d16)
            ic[sl] = s16
            vc[sl] = jnp.where(m, 0.0, 1.0)

    for b in range(2):
        ic, vc, sem = bufs[b]
        compute(b, ic, vc)
        pltpu.async_copy(vc, deg_sh.at[ic], sem, add=True)

    def body(j, carry):
        for b in range(2):
            i = 2 * j + 2 + b
            ic, vc, sem = bufs[b]
            pltpu.make_async_copy(vc, deg_sh.at[ic], sem).wait()
            compute(i, ic, vc)
            pltpu.async_copy(vc, deg_sh.at[ic], sem, add=True)
        return carry

    lax.fori_loop(0, ENFULL // 2 - 1, body, 0)
    for b in range(2):
        ic, vc, sem = bufs[b]
        pltpu.make_async_copy(vc, deg_sh.at[ic], sem).wait()

    off = ENFULL * ECH
    dsl = pl.ds(off, 16)
    s16 = srcb[dsl]
    d16 = dstb[dsl]
    m = s16 == d16
    ndb[dsl] = jnp.where(m, TRASH, d16)
    srct[...] = s16
    valt[...] = jnp.where(m, 0.0, 1.0)
    pltpu.sync_copy(valt, deg_sh.at[srct], add=True)

    pltpu.sync_copy(ndb, dstp_h.at[pl.ds(ebase, EPW)])

    plsc.subcore_barrier()
    pltpu.sync_copy(
        deg_sh.at[pl.ds(sid * RPT, RPT)],
        degp_h.at[pl.ds(cid * NACC + sid * RPT, RPT)],
    )


@functools.partial(
    pl.kernel,
    mesh=_mesh,
    out_type=jax.ShapeDtypeStruct((NC * NACC, D), jnp.float32),
    scratch_types=[
        pltpu.VMEM((EPW,), jnp.int32),
        pltpu.VMEM((CH,), jnp.int32),
        pltpu.VMEM((CH,), jnp.int32),
        pltpu.VMEM((CH,), jnp.int32),
        pltpu.VMEM((CH, D), jnp.float32),
        pltpu.VMEM((CH, D), jnp.float32),
        pltpu.VMEM((CH, D), jnp.float32),
        pltpu.VMEM((TAIL,), jnp.int32),
        pltpu.VMEM((4, D), jnp.float32),
        pltpu.VMEM_SHARED((NACC, D), jnp.float32),
        pltpu.SemaphoreType.DMA,
        pltpu.SemaphoreType.DMA,
        pltpu.SemaphoreType.DMA,
        pltpu.SemaphoreType.DMA,
        pltpu.SemaphoreType.DMA,
        pltpu.SemaphoreType.DMA,
        pltpu.SemaphoreType.DMA,
        pltpu.SemaphoreType.DMA,
        pltpu.SemaphoreType.DMA,
    ],
)
def _spmv(g_h, src_h, dstp_h, out_h,
          srcb, dc0, dc1, dc2, r0, r1, r2, dstt, zb, acc_sh,
          sg0, sg1, sg2, sd0, sd1, sd2, ss0, ss1, ss2):
    cid = lax.axis_index("c")
    sid = lax.axis_index("s")
    wid = sid * NC + cid
    ebase = wid * EPW

    for r in range(4):
        for j in range(D // 16):
            zb[r, pl.ds(j * 16, 16)] = jnp.zeros((16,), jnp.float32)

    pltpu.async_copy(src_h.at[pl.ds(ebase, EPW)], srcb, sg0)

    def zbody(i, carry):
        pltpu.sync_copy(zb, acc_sh.at[pl.ds(sid * RPT + i * 4, 4)])
        return carry

    lax.fori_loop(0, RPT // 4, zbody, 0)
    pltpu.make_async_copy(src_h.at[pl.ds(ebase, EPW)], srcb, sg0).wait()
    plsc.subcore_barrier()

    DC = (dc0, dc1, dc2)
    RW = (r0, r1, r2)
    SG = (sg0, sg1, sg2)
    SD = (sd0, sd1, sd2)
    SS = (ss0, ss1, ss2)

    def gather(i, b):
        off = i * CH
        pltpu.async_copy(dstp_h.at[pl.ds(ebase + off, CH)], DC[b], SD[b])
        pltpu.async_copy(g_h.at[srcb.at[pl.ds(off, CH)]], RW[b], SG[b])

    def wait_gather(i, b):
        off = i * CH
        pltpu.make_async_copy(dstp_h.at[pl.ds(ebase + off, CH)], DC[b],
                              SD[b]).wait()
        pltpu.make_async_copy(g_h.at[srcb.at[pl.ds(off, CH)]], RW[b],
                              SG[b]).wait()

    def scatter(b):
        pltpu.async_copy(RW[b], acc_sh.at[DC[b]], SS[b], add=True)

    def wait_scatter(b):
        pltpu.make_async_copy(RW[b], acc_sh.at[DC[b]], SS[b]).wait()

    gather(0, 0)
    gather(1, 1)
    gather(2, 2)
    wait_gather(0, 0)
    scatter(0)
    wait_scatter(0)
    gather(3, 0)
    wait_gather(1, 1)
    scatter(1)
    wait_scatter(1)
    gather(4, 1)
    wait_gather(2, 2)
    scatter(2)

    def body(j, carry):
        for b in range(3):
            i = 3 * j + b
            bt = (b + 2) % 3
            wait_scatter(bt)
            gather(i + 2, bt)
            wait_gather(i, b)
            scatter(b)
        return carry

    lax.fori_loop(1, (NFULL - 2) // 3, body, 0)

    wait_scatter(2)
    wait_gather(NFULL - 2, 0)
    scatter(0)
    wait_scatter(0)
    wait_gather(NFULL - 1, 1)
    scatter(1)
    wait_scatter(1)

    off = NFULL * CH
    pltpu.async_copy(dstp_h.at[pl.ds(ebase + off, TAIL)], dstt, sd0).wait()
    pltpu.async_copy(g_h.at[srcb.at[pl.ds(off, TAIL)]],
                     r0.at[pl.ds(0, TAIL)], sg0).wait()
    pltpu.sync_copy(r0.at[pl.ds(0, TAIL)], acc_sh.at[dstt], add=True)

    plsc.subcore_barrier()
    pltpu.sync_copy(
        acc_sh.at[pl.ds(sid * RPT, RPT)],
        out_h.at[pl.ds(cid * NACC + sid * RPT, RPT)],
    )


BLK = 1000
GRID = N // BLK


def _prescale_body(degT_ref, x_ref, dinv_ref, g_ref):
    deg = degT_ref[...]
    d = deg[:, 0:1] + deg[:, 1:2]
    dinv = jnp.where(d > 0.0, lax.rsqrt(d), 0.0)
    dinv_ref[...] = dinv
    g_ref[...] = x_ref[...] * dinv


def _prescale(degT, x):
    return pl.pallas_call(
        _prescale_body,
        grid=(GRID,),
        in_specs=[
            pl.BlockSpec((BLK, NC), lambda i: (i, 0)),
            pl.BlockSpec((BLK, D), lambda i: (i, 0)),
        ],
        out_specs=[
            pl.BlockSpec((BLK, 1), lambda i: (i, 0)),
            pl.BlockSpec((BLK, D), lambda i: (i, 0)),
        ],
        out_shape=[
            jax.ShapeDtypeStruct((N, 1), jnp.float32),
            jax.ShapeDtypeStruct((N, D), jnp.float32),
        ],
    )(degT, x)


def _mid_body(dinv_ref, s1a_ref, s1b_ref, g2_ref):
    d = dinv_ref[...]
    g2_ref[...] = -(d * d) * (s1a_ref[...] + s1b_ref[...])


def _mid(dinv, sa, sb):
    return pl.pallas_call(
        _mid_body,
        grid=(GRID,),
        in_specs=[
            pl.BlockSpec((BLK, 1), lambda i: (i, 0)),
            pl.BlockSpec((BLK, D), lambda i: (i, 0)),
            pl.BlockSpec((BLK, D), lambda i: (i, 0)),
        ],
        out_specs=pl.BlockSpec((BLK, D), lambda i: (i, 0)),
        out_shape=jax.ShapeDtypeStruct((N, D), jnp.float32),
    )(dinv, sa, sb)


def _midb_body(x_ref, dinv_ref, s1a, s1b, w_ref, b_ref, o_ref):
    d = dinv_ref[...]
    tx1 = -d * (s1a[...] + s1b[...])
    w02 = w_ref[0] - w_ref[2]
    o_ref[...] = (jnp.dot(x_ref[...], w02, preferred_element_type=jnp.float32)
                  + jnp.dot(tx1, w_ref[1], preferred_element_type=jnp.float32)
                  + b_ref[...])


def _midb(x, dinv, s1a, s1b, W, b2):
    return pl.pallas_call(
        _midb_body,
        grid=(GRID,),
        in_specs=[
            pl.BlockSpec((BLK, D), lambda i: (i, 0)),
            pl.BlockSpec((BLK, 1), lambda i: (i, 0)),
            pl.BlockSpec((BLK, D), lambda i: (i, 0)),
            pl.BlockSpec((BLK, D), lambda i: (i, 0)),
            pl.BlockSpec((3, D, D), lambda i: (0, 0, 0)),
            pl.BlockSpec((1, D), lambda i: (0, 0)),
        ],
        out_specs=pl.BlockSpec((BLK, D), lambda i: (i, 0)),
        out_shape=jax.ShapeDtypeStruct((N, D), jnp.float32),
    )(x, dinv, s1a, s1b, W, b2)


def _mmn_body(op1_ref, dinv_ref, s2a, s2b, w_ref,
              gam_ref, bet_ref, y_ref, acc, oall):
    p = pl.program_id(0)
    i = pl.program_id(1)

    @pl.when(p == 0)
    def _compute():
        d = dinv_ref[...]
        tx2s = -2.0 * d * (s2a[...] + s2b[...])
        o = (op1_ref[...]
             + jnp.dot(tx2s, w_ref[2], preferred_element_type=jnp.float32))
        oall[pl.ds(i * BLK, BLK), :] = o

        @pl.when(i == 0)
        def _init():
            acc[...] = jnp.zeros_like(acc)

        acc[0:1, :] += jnp.sum(o, axis=0, keepdims=True)
        acc[1:2, :] += jnp.sum(o * o, axis=0, keepdims=True)

    @pl.when(p == 1)
    def _normalize():
        st = acc[...]
        mean = st[0:1, :] * (1.0 / N)
        var = st[1:2, :] * (1.0 / N) - mean * mean
        scale = lax.rsqrt(var + EPS) * gam_ref[...]
        o = oall[pl.ds(i * BLK, BLK), :]
        y = (o - mean) * scale + bet_ref[...]
        y_ref[...] = jnp.where(y >= 0.0, y, ALPHA * y)


def _mmn(op1, dinv, s2a, s2b, W, gam2, bet2):
    blk = lambda p, i: (i * (1 - p), 0)
    return pl.pallas_call(
        _mmn_body,
        grid=(2, GRID),
        in_specs=[
            pl.BlockSpec((BLK, D), blk),
            pl.BlockSpec((BLK, 1), blk),
            pl.BlockSpec((BLK, D), blk),
            pl.BlockSpec((BLK, D), blk),
            pl.BlockSpec((3, D, D), lambda p, i: (0, 0, 0)),
            pl.BlockSpec((1, D), lambda p, i: (0, 0)),
            pl.BlockSpec((1, D), lambda p, i: (0, 0)),
        ],
        out_specs=pl.BlockSpec((BLK, D), lambda p, i: (i * p, 0)),
        out_shape=jax.ShapeDtypeStruct((N, D), jnp.float32),
        scratch_shapes=[
            pltpu.VMEM((2, D), jnp.float32),
            pltpu.VMEM((N, D), jnp.float32),
        ],
    )(op1, dinv, s2a, s2b, W, gam2, bet2)


@jax.jit
def kernel(x, edge_idx, W, b, gamma, beta):
    src = edge_idx[0]
    dst = edge_idx[1]
    degp, dstp = _edge_prep(src, dst)
    degT = jnp.transpose(degp.reshape(NC, NACC)[:, :N])
    dinv, g1 = _prescale(degT, x)
    s1 = _spmv(g1, src, dstp)
    s1a, s1b = s1[:N], s1[NACC:NACC + N]
    g2 = _mid(dinv, s1a, s1b)
    s2 = _spmv(g2, src, dstp)
    op1 = _midb(x, dinv, s1a, s1b, W, b.reshape(1, D))
    s2a, s2b = s2[:N], s2[NACC:NACC + N]
    return _mmn(op1, dinv, s2a, s2b, W,
                gamma.reshape(1, D), beta.reshape(1, D))
